# Initial kernel scaffold; baseline (speedup 1.0000x reference)
import jax
import jax.numpy as jnp
from jax.experimental import pallas as pl


def kernel(x, tables, W_pos, W_fut, W_isfut):
    raise NotImplementedError("write your pallas kernel here")


# speedup vs baseline: 3.9353x; 3.9353x over previous
import jax
import jax.numpy as jnp
from jax import lax
from jax.experimental import pallas as pl
from jax.experimental.pallas import tpu as pltpu
from jax.experimental.pallas import tpu_sc as plsc

SEQ_LEN = 200
LAG = 50
D_MODEL = 32
NVARS = 8
VOCAB = 100000
B = 1024

NTOK = B * SEQ_LEN
NC, NS = 2, 16
NW = NC * NS
TOK_PER_W = NTOK // NW
B_PER_W = B // NW
CHUNK = 128
N_CHUNKS = TOK_PER_W // CHUNK
LANES = 16
PAST = SEQ_LEN - LAG
FUT_OFF = 200
ISF_OFF = 256


def _row16(ref_flat, row, col0):
    base = row * D_MODEL + col0
    return plsc.load_gather(ref_flat, [lax.iota(jnp.int32, LANES) + base])


def _body(x_hbm, tbl_hbm, wpos_hbm, wfut_hbm, wisfut_hbm, out_hbm,
          xv, idx2d, rows, pos_b, small_b, sem):
    wid = lax.axis_index("s") * NC + lax.axis_index("c")
    tok0 = wid * TOK_PER_W

    pltpu.sync_copy(wpos_hbm, small_b.at[pl.ds(0, SEQ_LEN * D_MODEL)])
    pltpu.sync_copy(wfut_hbm,
                    small_b.at[pl.ds(FUT_OFF * D_MODEL,
                                     (LAG + 1) * D_MODEL)])
    pltpu.sync_copy(wisfut_hbm,
                    small_b.at[pl.ds(ISF_OFF * D_MODEL, 2 * D_MODEL)])

    def fill(t, _):
        j1 = FUT_OFF + jnp.maximum(0, t - (PAST - 1))
        j2 = ISF_OFF + (t >= PAST).astype(jnp.int32)
        for half in range(2):
            c0 = half * LANES
            pos_b[t, pl.ds(c0, LANES)] = _row16(small_b, t, c0)
            pos_b[SEQ_LEN + t, pl.ds(c0, LANES)] = _row16(small_b, j1, c0)
            pos_b[2 * SEQ_LEN + t, pl.ds(c0, LANES)] = _row16(small_b, j2,
                                                              c0)
        return _

    lax.fori_loop(0, SEQ_LEN, fill, 0)

    def pos_out(i, _):
        b = wid * B_PER_W + i
        for k in range(3):
            pltpu.sync_copy(pos_b.at[pl.ds(k * SEQ_LEN, SEQ_LEN)],
                            out_hbm.at[pl.ds(b * SEQ_LEN, SEQ_LEN),
                                       NVARS + k])
        return _

    lax.fori_loop(0, B_PER_W, pos_out, 0)

    def chunk_body(c, _):
        base = tok0 + c * CHUNK
        pltpu.sync_copy(x_hbm.at[pl.ds(base * NVARS, CHUNK * NVARS)], xv)

        for i in range(NVARS):
            for v in range(CHUNK // LANES):
                tvec = lax.iota(jnp.int32, LANES) + (v * LANES)
                raw = plsc.load_gather(xv, [tvec * NVARS + i])
                idx2d[i, pl.ds(v * LANES, LANES)] = raw + (i * VOCAB)

        copies = []
        for i in range(NVARS):
            copies.append(pltpu.async_copy(
                tbl_hbm.at[idx2d.at[i]], rows.at[i], sem))
        for cp in copies:
            cp.wait()

        for i in range(NVARS):
            pltpu.sync_copy(rows.at[i],
                            out_hbm.at[pl.ds(base, CHUNK), i])
        return _

    lax.fori_loop(0, N_CHUNKS, chunk_body, 0, unroll=False)


@jax.jit
def _run(x_flat, tbl_flat, W_pos, W_fut, W_isfut):
    mesh = plsc.VectorSubcoreMesh(core_axis_name="c", subcore_axis_name="s")
    out = pl.kernel(
        _body,
        out_type=jax.ShapeDtypeStruct((NTOK, NVARS + 3, D_MODEL),
                                      jnp.float32),
        mesh=mesh,
        compiler_params=pltpu.CompilerParams(needs_layout_passes=False,
                                             use_tc_tiling_on_sc=False),
        scratch_types=[
            pltpu.VMEM((CHUNK * NVARS,), jnp.int32),
            pltpu.VMEM((NVARS, CHUNK), jnp.int32),
            pltpu.VMEM((NVARS, CHUNK, D_MODEL), jnp.float32),
            pltpu.VMEM((3 * SEQ_LEN, D_MODEL), jnp.float32),
            pltpu.VMEM(((ISF_OFF + 8) * D_MODEL,), jnp.float32),
            pltpu.SemaphoreType.DMA,
        ],
    )(x_flat, tbl_flat, W_pos, W_fut, W_isfut)
    return out


def kernel(x, tables, W_pos, W_fut, W_isfut):
    x_flat = x.astype(jnp.int32).reshape(NTOK * NVARS)
    tbl_flat = tables.reshape(NVARS * VOCAB, D_MODEL)
    out = _run(x_flat, tbl_flat, W_pos.reshape(-1), W_fut.reshape(-1),
               W_isfut.reshape(-1))
    return out.reshape(B, SEQ_LEN, NVARS + 3, D_MODEL)


# speedup vs baseline: 3.9926x; 1.0145x over previous
import jax
import jax.numpy as jnp
from jax import lax
from jax.experimental import pallas as pl
from jax.experimental.pallas import tpu as pltpu
from jax.experimental.pallas import tpu_sc as plsc

SEQ_LEN = 200
LAG = 50
D_MODEL = 32
NVARS = 8
VOCAB = 100000
B = 1024

NC, NS = 2, 16
NW = NC * NS
B_PER_W = B // NW
LANES = 16
PAST = SEQ_LEN - LAG
NVEC = (SEQ_LEN + LANES - 1) // LANES
IDXW = NVEC * LANES
FUT_OFF = 200
ISF_OFF = 256


def _row16(ref_flat, row, col0):
    base = row * D_MODEL + col0
    return plsc.load_gather(ref_flat, [lax.iota(jnp.int32, LANES) + base])


def _body(x_hbm, tbl_hbm, wpos_hbm, wfut_hbm, wisfut_hbm, out_hbm,
          xv, idx2d, rows, pos_b, small_b, sem):
    wid = lax.axis_index("s") * NC + lax.axis_index("c")

    pltpu.sync_copy(wpos_hbm, small_b.at[pl.ds(0, SEQ_LEN * D_MODEL)])
    pltpu.sync_copy(wfut_hbm,
                    small_b.at[pl.ds(FUT_OFF * D_MODEL,
                                     (LAG + 1) * D_MODEL)])
    pltpu.sync_copy(wisfut_hbm,
                    small_b.at[pl.ds(ISF_OFF * D_MODEL, 2 * D_MODEL)])

    def fill(t, _):
        j1 = FUT_OFF + jnp.maximum(0, t - (PAST - 1))
        j2 = ISF_OFF + (t >= PAST).astype(jnp.int32)
        for half in range(2):
            c0 = half * LANES
            pos_b[t, pl.ds(c0, LANES)] = _row16(small_b, t, c0)
            pos_b[SEQ_LEN + t, pl.ds(c0, LANES)] = _row16(small_b, j1, c0)
            pos_b[2 * SEQ_LEN + t, pl.ds(c0, LANES)] = _row16(small_b, j2,
                                                              c0)
        return _

    lax.fori_loop(0, SEQ_LEN, fill, 0)

    def batch_body(bi, _):
        b = wid * B_PER_W + bi
        pltpu.sync_copy(x_hbm.at[b], xv)

        ivecs = [jnp.full((LANES,), i, dtype=jnp.int32)
                 for i in range(NVARS)]
        for i in range(NVARS):
            for v in range(NVEC):
                tvec = lax.iota(jnp.int32, LANES) + (v * LANES)
                if v == NVEC - 1:
                    tvec = jnp.minimum(tvec, SEQ_LEN - 1)
                raw = plsc.load_gather(xv, [tvec, ivecs[i]])
                idx2d[i, pl.ds(v * LANES, LANES)] = raw

        copies = []
        for i in range(NVARS):
            copies.append(pltpu.async_copy(
                tbl_hbm.at[i].at[idx2d.at[i].at[pl.ds(0, 128)]],
                rows.at[i].at[pl.ds(0, 128)], sem))
            copies.append(pltpu.async_copy(
                tbl_hbm.at[i].at[idx2d.at[i].at[pl.ds(128, SEQ_LEN - 128)]],
                rows.at[i].at[pl.ds(128, SEQ_LEN - 128)], sem))
        for cp in copies:
            cp.wait()

        for i in range(NVARS):
            pltpu.sync_copy(rows.at[i], out_hbm.at[b, :, i])
        for k in range(3):
            pltpu.sync_copy(pos_b.at[pl.ds(k * SEQ_LEN, SEQ_LEN)],
                            out_hbm.at[b, :, NVARS + k])
        return _

    lax.fori_loop(0, B_PER_W, batch_body, 0, unroll=False)


@jax.jit
def _run(x, tables, W_pos, W_fut, W_isfut):
    mesh = plsc.VectorSubcoreMesh(core_axis_name="c", subcore_axis_name="s")
    out = pl.kernel(
        _body,
        out_type=jax.ShapeDtypeStruct((B, SEQ_LEN, NVARS + 3, D_MODEL),
                                      jnp.float32),
        mesh=mesh,
        compiler_params=pltpu.CompilerParams(needs_layout_passes=False,
                                             use_tc_tiling_on_sc=False),
        scratch_types=[
            pltpu.VMEM((SEQ_LEN, NVARS), jnp.int32),
            pltpu.VMEM((NVARS, IDXW), jnp.int32),
            pltpu.VMEM((NVARS, SEQ_LEN, D_MODEL), jnp.float32),
            pltpu.VMEM((3 * SEQ_LEN, D_MODEL), jnp.float32),
            pltpu.VMEM(((ISF_OFF + 8) * D_MODEL,), jnp.float32),
            pltpu.SemaphoreType.DMA,
        ],
    )(x, tables, W_pos, W_fut, W_isfut)
    return out


def kernel(x, tables, W_pos, W_fut, W_isfut):
    return _run(x.astype(jnp.int32), tables, W_pos.reshape(-1),
                W_fut.reshape(-1), W_isfut.reshape(-1))


# speedup vs baseline: 4.4367x; 1.1112x over previous
import jax
import jax.numpy as jnp
from jax import lax
from jax.experimental import pallas as pl
from jax.experimental.pallas import tpu as pltpu
from jax.experimental.pallas import tpu_sc as plsc

SEQ_LEN = 200
LAG = 50
D_MODEL = 32
NVARS = 8
VOCAB = 100000
B = 1024

NC, NS = 2, 16
NW = NC * NS
LANES = 16
PAST = SEQ_LEN - LAG
NBC = B // 128
NUNITS = SEQ_LEN * NBC
U_PER_W = NUNITS // NW
NSLOT = NVARS + 3
KD = NSLOT * (D_MODEL // 8)
FUT_OFF = 200
ISF_OFF = 256


def _row16(ref_flat, row, col0):
    base = row * D_MODEL + col0
    return plsc.load_gather(ref_flat, [lax.iota(jnp.int32, LANES) + base])


def _body(x_hbm, tbl_hbm, wpos_hbm, wfut_hbm, wisfut_hbm, out_hbm,
          xv, rows, obuf, pos_b, small_b, sem):
    wid = lax.axis_index("s") * NC + lax.axis_index("c")

    pltpu.sync_copy(wpos_hbm, small_b.at[pl.ds(0, SEQ_LEN * D_MODEL)])
    pltpu.sync_copy(wfut_hbm,
                    small_b.at[pl.ds(FUT_OFF * D_MODEL,
                                     (LAG + 1) * D_MODEL)])
    pltpu.sync_copy(wisfut_hbm,
                    small_b.at[pl.ds(ISF_OFF * D_MODEL, 2 * D_MODEL)])

    def fill(t, _):
        j1 = FUT_OFF + jnp.maximum(0, t - (PAST - 1))
        j2 = ISF_OFF + (t >= PAST).astype(jnp.int32)
        for half in range(2):
            c0 = half * LANES
            pos_b[t, pl.ds(c0, LANES)] = _row16(small_b, t, c0)
            pos_b[SEQ_LEN + t, pl.ds(c0, LANES)] = _row16(small_b, j1, c0)
            pos_b[2 * SEQ_LEN + t, pl.ds(c0, LANES)] = _row16(small_b, j2,
                                                              c0)
        return _

    lax.fori_loop(0, SEQ_LEN, fill, 0)

    def unit_body(ui, _):
        u = wid * U_PER_W + ui
        t = u // NBC
        bc = u - t * NBC

        pltpu.sync_copy(x_hbm.at[t, bc], xv)

        copies = []
        for i in range(NVARS):
            copies.append(pltpu.async_copy(
                tbl_hbm.at[i].at[xv.at[i]],
                rows.at[pl.ds(i * 128, 128)], sem))

        for k in range(3):
            prow = k * SEQ_LEN + t
            for d in range(D_MODEL):
                vals = plsc.load_gather(
                    pos_b, [jnp.full((LANES,), prow, dtype=jnp.int32),
                            jnp.full((LANES,), d, dtype=jnp.int32)])
                r = (NVARS + k) * (D_MODEL // 8) + d // 8
                dl = d % 8
                for bv in range(8):
                    obuf[r, dl, pl.ds(bv * LANES, LANES)] = vals

        for cp in copies:
            cp.wait()

        for i in range(NVARS):
            for d in range(D_MODEL):
                r = i * (D_MODEL // 8) + d // 8
                dl = d % 8
                dvec = jnp.full((LANES,), d, dtype=jnp.int32)
                for bv in range(8):
                    rowvec = (lax.iota(jnp.int32, LANES)
                              + (i * 128 + bv * LANES))
                    vals = plsc.load_gather(rows, [rowvec, dvec])
                    obuf[r, dl, pl.ds(bv * LANES, LANES)] = vals

        pltpu.sync_copy(obuf, out_hbm.at[t, :, bc])
        return _

    lax.fori_loop(0, U_PER_W, unit_body, 0, unroll=False)


@jax.jit
def _run(x4, tables, W_pos, W_fut, W_isfut):
    mesh = plsc.VectorSubcoreMesh(core_axis_name="c", subcore_axis_name="s")
    out = pl.kernel(
        _body,
        out_type=jax.ShapeDtypeStruct((SEQ_LEN, KD, NBC, 8, 128),
                                      jnp.float32),
        mesh=mesh,
        compiler_params=pltpu.CompilerParams(needs_layout_passes=False,
                                             use_tc_tiling_on_sc=False),
        scratch_types=[
            pltpu.VMEM((NVARS, 128), jnp.int32),
            pltpu.VMEM((NVARS * 128, D_MODEL), jnp.float32),
            pltpu.VMEM((KD, 8, 128), jnp.float32),
            pltpu.VMEM((3 * SEQ_LEN, D_MODEL), jnp.float32),
            pltpu.VMEM(((ISF_OFF + 8) * D_MODEL,), jnp.float32),
            pltpu.SemaphoreType.DMA,
        ],
    )(x4, tables, W_pos, W_fut, W_isfut)
    return out


def kernel(x, tables, W_pos, W_fut, W_isfut):
    x4 = x.astype(jnp.int32).reshape(NBC, 128, SEQ_LEN, NVARS)
    x4 = x4.transpose((2, 0, 3, 1))
    out = _run(x4, tables, W_pos.reshape(-1), W_fut.reshape(-1),
               W_isfut.reshape(-1))
    out = out.transpose((2, 4, 0, 1, 3)).reshape(B, SEQ_LEN, NSLOT,
                                                 D_MODEL)
    return out


# speedup vs baseline: 5.7182x; 1.2888x over previous
import jax
import jax.numpy as jnp
from jax import lax
from jax.experimental import pallas as pl
from jax.experimental.pallas import tpu as pltpu
from jax.experimental.pallas import tpu_sc as plsc

SEQ_LEN = 200
LAG = 50
D_MODEL = 32
NVARS = 8
VOCAB = 100000
B = 1024

NC, NS = 2, 16
NW = NC * NS
LANES = 16
PAST = SEQ_LEN - LAG
NBC = B // 128
NUNITS = SEQ_LEN * NBC
U_PER_W = NUNITS // NW
NSLOT = NVARS + 3
KD = NSLOT * (D_MODEL // 8)
FUT_OFF = 200
ISF_OFF = 256


def _body(x_hbm, tbl_hbm, wpos_hbm, wfut_hbm, wisfut_hbm, out_hbm,
          xvt, rows, obuf, small_b, gsem, wsem):
    wid = lax.axis_index("s") * NC + lax.axis_index("c")

    pltpu.sync_copy(wpos_hbm, small_b.at[pl.ds(0, SEQ_LEN * D_MODEL)])
    pltpu.sync_copy(wfut_hbm,
                    small_b.at[pl.ds(FUT_OFF * D_MODEL,
                                     (LAG + 1) * D_MODEL)])
    pltpu.sync_copy(wisfut_hbm,
                    small_b.at[pl.ds(ISF_OFF * D_MODEL, 2 * D_MODEL)])

    rvecs = [lax.iota(jnp.int32, LANES) + bv * LANES for bv in range(8)]

    def unit_body(ui, tprev):
        u = wid * U_PER_W + ui
        t = u // NBC
        bc = u - t * NBC
        tchange = t != tprev

        @pl.when(tchange)
        def _():
            pltpu.sync_copy(x_hbm.at[t], xvt)

        copies = []
        for i in range(NVARS):
            copies.append(pltpu.async_copy(
                tbl_hbm.at[i].at[xvt.at[bc].at[i]],
                rows.at[pl.ds(i * 128, 128)], gsem))

        @pl.when(ui > 0)
        def _():
            pltpu.make_async_copy(obuf, out_hbm.at[t, :, bc], wsem).wait()

        @pl.when(tchange)
        def _():
            j1 = FUT_OFF + jnp.maximum(0, t - (PAST - 1))
            j2 = ISF_OFF + (t >= PAST).astype(jnp.int32)
            for k, jr in ((0, t), (1, j1), (2, j2)):
                vals = []
                for d in range(D_MODEL):
                    vals.append(plsc.load_gather(
                        small_b,
                        [jnp.full((LANES,), jr * D_MODEL + d,
                                  dtype=jnp.int32)]))
                for d in range(D_MODEL):
                    r = (NVARS + k) * (D_MODEL // 8) + d // 8
                    for bv in range(8):
                        obuf[r, d % 8, pl.ds(bv * LANES, LANES)] = vals[d]

        for cp in copies:
            cp.wait()

        for i in range(NVARS):
            rowvecs = [rv + i * 128 for rv in rvecs]
            for dr in range(D_MODEL // 8):
                pairs = [(dr * 8 + dd, bv) for dd in range(8)
                         for bv in range(8)]
                for w0 in range(0, 64, 16):
                    wave = pairs[w0:w0 + 16]
                    vals = [plsc.load_gather(
                        rows, [rowvecs[bv],
                               jnp.full((LANES,), d, dtype=jnp.int32)])
                        for d, bv in wave]
                    for (d, bv), v in zip(wave, vals):
                        obuf[i * 4 + dr, d % 8,
                             pl.ds(bv * LANES, LANES)] = v

        pltpu.async_copy(obuf, out_hbm.at[t, :, bc], wsem)
        return t

    lax.fori_loop(0, U_PER_W, unit_body, -1, unroll=False)
    pltpu.make_async_copy(obuf, out_hbm.at[0, :, 0], wsem).wait()


@jax.jit
def _run(x4, tables, W_pos, W_fut, W_isfut):
    mesh = plsc.VectorSubcoreMesh(core_axis_name="c", subcore_axis_name="s")
    out = pl.kernel(
        _body,
        out_type=jax.ShapeDtypeStruct((SEQ_LEN, KD, NBC, 8, 128),
                                      jnp.float32),
        mesh=mesh,
        compiler_params=pltpu.CompilerParams(needs_layout_passes=False,
                                             use_tc_tiling_on_sc=False),
        scratch_types=[
            pltpu.VMEM((NBC, NVARS, 128), jnp.int32),
            pltpu.VMEM((NVARS * 128, D_MODEL), jnp.float32),
            pltpu.VMEM((KD, 8, 128), jnp.float32),
            pltpu.VMEM(((ISF_OFF + 8) * D_MODEL,), jnp.float32),
            pltpu.SemaphoreType.DMA,
            pltpu.SemaphoreType.DMA,
        ],
    )(x4, tables, W_pos, W_fut, W_isfut)
    return out


def kernel(x, tables, W_pos, W_fut, W_isfut):
    x4 = x.astype(jnp.int32).reshape(NBC, 128, SEQ_LEN, NVARS)
    x4 = x4.transpose((2, 0, 3, 1))
    out = _run(x4, tables, W_pos.reshape(-1), W_fut.reshape(-1),
               W_isfut.reshape(-1))
    out = out.transpose((2, 4, 0, 1, 3)).reshape(B, SEQ_LEN, NSLOT,
                                                 D_MODEL)
    return out


# speedup vs baseline: 10.5993x; 1.8536x over previous
import jax
import jax.numpy as jnp
from jax import lax
from jax.experimental import pallas as pl
from jax.experimental.pallas import tpu as pltpu
from jax.experimental.pallas import tpu_sc as plsc

SEQ_LEN = 200
LAG = 50
D_MODEL = 32
NVARS = 8
VOCAB = 100000
B = 1024

NC, NS = 2, 16
NW = NC * NS
LANES = 16
PAST = SEQ_LEN - LAG
NBC = B // 128
NUNITS = SEQ_LEN * NBC
U_PER_W = NUNITS // NW
NSLOT = NVARS + 3
KD = NSLOT * (D_MODEL // 8)
FUT_OFF = 200
ISF_OFF = 256


def _body(x_hbm, tbl_hbm, wpos_hbm, wfut_hbm, wisfut_hbm, out_hbm,
          xvt, rows, obuf, small_b, gsem, wsem):
    wid = lax.axis_index("s") * NC + lax.axis_index("c")

    pltpu.sync_copy(wpos_hbm, small_b.at[pl.ds(0, SEQ_LEN * D_MODEL)])
    pltpu.sync_copy(wfut_hbm,
                    small_b.at[pl.ds(FUT_OFF * D_MODEL,
                                     (LAG + 1) * D_MODEL)])
    pltpu.sync_copy(wisfut_hbm,
                    small_b.at[pl.ds(ISF_OFF * D_MODEL, 2 * D_MODEL)])

    iota = lax.iota(jnp.int32, LANES)
    rvecs = [iota + bv * LANES for bv in range(8)]
    diags = [lax.rem(iota + sft, jnp.int32(LANES)) for sft in range(LANES)]
    iota32 = iota * D_MODEL

    def unit_body(ui, tprev):
        u = wid * U_PER_W + ui
        t = u // NBC
        bc = u - t * NBC
        tchange = t != tprev

        @pl.when(tchange)
        def _():
            pltpu.sync_copy(x_hbm.at[t], xvt)

        copies = []
        for i in range(NVARS):
            copies.append(pltpu.async_copy(
                tbl_hbm.at[i].at[xvt.at[bc].at[i]],
                rows.at[pl.ds(i * 128, 128)], gsem))

        @pl.when(ui > 0)
        def _():
            pltpu.make_async_copy(obuf, out_hbm.at[t, :, bc], wsem).wait()

        @pl.when(tchange)
        def _():
            j1 = FUT_OFF + jnp.maximum(0, t - (PAST - 1))
            j2 = ISF_OFF + (t >= PAST).astype(jnp.int32)
            for k, jr in ((0, t), (1, j1), (2, j2)):
                vals = []
                for d in range(D_MODEL):
                    vals.append(plsc.load_gather(
                        small_b,
                        [jnp.full((LANES,), jr * D_MODEL + d,
                                  dtype=jnp.int32)]))
                for d in range(D_MODEL):
                    r = (NVARS + k) * (D_MODEL // 8) + d // 8
                    for bv in range(8):
                        obuf[r, d % 8, pl.ds(bv * LANES, LANES)] = vals[d]

        for cp in copies:
            cp.wait()

        def tr_body(v, _):
            i = v >> 3
            bv = v & 7
            rowv = iota + (bv * LANES + i * 128)
            blv = iota + bv * LANES
            for dh in range(2):
                for w0 in range(0, LANES, 8):
                    dvecs = [diags[sft] + dh * LANES
                             for sft in range(w0, w0 + 8)]
                    vals = [plsc.load_gather(rows, [rowv, dv])
                            for dv in dvecs]
                    for dv, val in zip(dvecs, vals):
                        rvec = (dv >> 3) + (i * 4)
                        dlv = dv & 7
                        plsc.store_scatter(obuf, [rvec, dlv, blv], val)
            return _

        lax.fori_loop(0, NVARS * 8, tr_body, 0, unroll=False)

        pltpu.async_copy(obuf, out_hbm.at[t, :, bc], wsem)
        return t

    lax.fori_loop(0, U_PER_W, unit_body, -1, unroll=False)
    pltpu.make_async_copy(obuf, out_hbm.at[0, :, 0], wsem).wait()


@jax.jit
def _run(x4, tables, W_pos, W_fut, W_isfut):
    mesh = plsc.VectorSubcoreMesh(core_axis_name="c", subcore_axis_name="s")
    out = pl.kernel(
        _body,
        out_type=jax.ShapeDtypeStruct((SEQ_LEN, KD, NBC, 8, 128),
                                      jnp.float32),
        mesh=mesh,
        compiler_params=pltpu.CompilerParams(needs_layout_passes=False,
                                             use_tc_tiling_on_sc=False),
        scratch_types=[
            pltpu.VMEM((NBC, NVARS, 128), jnp.int32),
            pltpu.VMEM((NVARS * 128, D_MODEL), jnp.float32),
            pltpu.VMEM((KD, 8, 128), jnp.float32),
            pltpu.VMEM(((ISF_OFF + 8) * D_MODEL,), jnp.float32),
            pltpu.SemaphoreType.DMA,
            pltpu.SemaphoreType.DMA,
        ],
    )(x4, tables, W_pos, W_fut, W_isfut)
    return out


def kernel(x, tables, W_pos, W_fut, W_isfut):
    x4 = x.astype(jnp.int32).reshape(NBC, 128, SEQ_LEN, NVARS)
    x4 = x4.transpose((2, 0, 3, 1))
    out = _run(x4, tables, W_pos.reshape(-1), W_fut.reshape(-1),
               W_isfut.reshape(-1))
    out = out.transpose((2, 4, 0, 1, 3)).reshape(B, SEQ_LEN, NSLOT,
                                                 D_MODEL)
    return out
